# Initial kernel scaffold; baseline (speedup 1.0000x reference)
import jax
import jax.numpy as jnp
from jax.experimental import pallas as pl


def kernel(input_ids, embed_table):
    raise NotImplementedError("write your pallas kernel here")


# speedup vs baseline: 1.4438x; 1.4438x over previous
import functools

import jax
import jax.numpy as jnp
from jax import lax
from jax.experimental import pallas as pl
from jax.experimental.pallas import tpu as pltpu
from jax.experimental.pallas import tpu_sc as plsc

VOCAB = 151936
D_MODEL = 2048
NUM_TOKENS = 4 * 2048

_NC = 2
_NS = 16
_NW = _NC * _NS

_B_PER_W = NUM_TOKENS // _NW
_CHUNK = 16
_N_CHUNKS = _B_PER_W // _CHUNK


def _gather_body(ids_hbm, table_hbm, out_hbm, idx_v, rows_v, gsem, ssem):
    wid = lax.axis_index("s") * _NC + lax.axis_index("c")
    base = wid * _B_PER_W
    pltpu.sync_copy(ids_hbm.at[pl.ds(base, _B_PER_W)], idx_v)

    def step(i):
        buf = rows_v.at[i % 2]
        idx = idx_v.at[pl.ds(i * _CHUNK, _CHUNK)]
        pltpu.async_copy(table_hbm.at[idx], buf, gsem).wait()
        pltpu.async_copy(buf, out_hbm.at[pl.ds(base + i * _CHUNK, _CHUNK)],
                         ssem).wait()

    pl.loop(0, _N_CHUNKS)(step)


@functools.partial(
    pl.kernel,
    out_type=jax.ShapeDtypeStruct((NUM_TOKENS, D_MODEL), jnp.float32),
    mesh=plsc.VectorSubcoreMesh(core_axis_name="c", subcore_axis_name="s"),
    scratch_types=[
        pltpu.VMEM((_B_PER_W,), jnp.int32),
        pltpu.VMEM((2, _CHUNK, D_MODEL), jnp.float32),
        pltpu.SemaphoreType.DMA,
        pltpu.SemaphoreType.DMA,
    ],
)
def _sc_gather(ids_hbm, table_hbm, out_hbm, idx_v, rows_v, gsem, ssem):
    _gather_body(ids_hbm, table_hbm, out_hbm, idx_v, rows_v, gsem, ssem)


def kernel(input_ids, embed_table):
    ids_flat = jnp.reshape(input_ids, (NUM_TOKENS,)).astype(jnp.int32)
    out = _sc_gather(ids_flat, embed_table)
    return jnp.reshape(out, (*input_ids.shape, D_MODEL))


# speedup vs baseline: 1.6835x; 1.1661x over previous
import functools

import jax
import jax.numpy as jnp
from jax import lax
from jax.experimental import pallas as pl
from jax.experimental.pallas import tpu as pltpu
from jax.experimental.pallas import tpu_sc as plsc

VOCAB = 151936
D_MODEL = 2048
NUM_TOKENS = 4 * 2048

_NC = 2
_NS = 16
_NW = _NC * _NS

_B_PER_W = NUM_TOKENS // _NW
_CHUNK = 16
_N_CHUNKS = _B_PER_W // _CHUNK


def _gather_body(ids_hbm, table_hbm, out_hbm, idx_v, rows_v,
                 gsem0, gsem1, ssem0, ssem1):
    wid = lax.axis_index("s") * _NC + lax.axis_index("c")
    base = wid * _B_PER_W
    pltpu.sync_copy(ids_hbm.at[pl.ds(base, _B_PER_W)], idx_v)

    gsems = (gsem0, gsem1)
    ssems = (ssem0, ssem1)

    def gather(i, b):
        idx = idx_v.at[pl.ds(i * _CHUNK, _CHUNK)]
        pltpu.async_copy(table_hbm.at[idx], rows_v.at[b], gsems[b])

    gather(0, 0)

    def pair(g):
        for b in range(2):
            i = g + b
            other = 1 - b
            @pl.when(i > 0)
            def _():
                pltpu.make_async_copy(
                    rows_v.at[other],
                    out_hbm.at[pl.ds(base, _CHUNK)],
                    ssems[other]).wait()

            @pl.when(i + 1 < _N_CHUNKS)
            def _():
                gather(i + 1, other)

            pltpu.make_async_copy(
                table_hbm.at[pl.ds(0, _CHUNK)], rows_v.at[b],
                gsems[b]).wait()
            pltpu.async_copy(
                rows_v.at[b],
                out_hbm.at[pl.ds(base + i * _CHUNK, _CHUNK)],
                ssems[b])

    pl.loop(0, _N_CHUNKS, step=2)(pair)

    lastb = (_N_CHUNKS - 1) % 2
    pltpu.make_async_copy(
        rows_v.at[lastb], out_hbm.at[pl.ds(base, _CHUNK)],
        ssems[lastb]).wait()


@functools.partial(
    pl.kernel,
    out_type=jax.ShapeDtypeStruct((NUM_TOKENS, D_MODEL), jnp.float32),
    mesh=plsc.VectorSubcoreMesh(core_axis_name="c", subcore_axis_name="s"),
    scratch_types=[
        pltpu.VMEM((_B_PER_W,), jnp.int32),
        pltpu.VMEM((2, _CHUNK, D_MODEL), jnp.float32),
        pltpu.SemaphoreType.DMA,
        pltpu.SemaphoreType.DMA,
        pltpu.SemaphoreType.DMA,
        pltpu.SemaphoreType.DMA,
    ],
)
def _sc_gather(ids_hbm, table_hbm, out_hbm, idx_v, rows_v,
               gsem0, gsem1, ssem0, ssem1):
    _gather_body(ids_hbm, table_hbm, out_hbm, idx_v, rows_v,
                 gsem0, gsem1, ssem0, ssem1)


def kernel(input_ids, embed_table):
    ids_flat = jnp.reshape(input_ids, (NUM_TOKENS,)).astype(jnp.int32)
    out = _sc_gather(ids_flat, embed_table)
    return jnp.reshape(out, (*input_ids.shape, D_MODEL))


# speedup vs baseline: 1.6861x; 1.0015x over previous
import functools

import jax
import jax.numpy as jnp
from jax import lax
from jax.experimental import pallas as pl
from jax.experimental.pallas import tpu as pltpu
from jax.experimental.pallas import tpu_sc as plsc

VOCAB = 151936
D_MODEL = 2048
NUM_TOKENS = 4 * 2048

_NC = 2
_NS = 16
_NW = _NC * _NS

_B_PER_W = NUM_TOKENS // _NW
_CHUNK = 8
_NBUF = 4
_AHEAD = _NBUF - 2
_N_CHUNKS = _B_PER_W // _CHUNK


def _gather_body(ids_hbm, table_hbm, out_hbm, idx_v, rows_v, *sems):
    gsems = sems[:_NBUF]
    ssems = sems[_NBUF:]
    wid = lax.axis_index("s") * _NC + lax.axis_index("c")
    base = wid * _B_PER_W
    pltpu.sync_copy(ids_hbm.at[pl.ds(base, _B_PER_W)], idx_v)

    def start_gather(i, b):
        idx = idx_v.at[pl.ds(i * _CHUNK, _CHUNK)]
        pltpu.async_copy(table_hbm.at[idx], rows_v.at[b], gsems[b])

    def wait_gather(b):
        pltpu.make_async_copy(
            table_hbm.at[pl.ds(0, _CHUNK)], rows_v.at[b], gsems[b]).wait()

    def start_scatter(i, b):
        pltpu.async_copy(
            rows_v.at[b], out_hbm.at[pl.ds(base + i * _CHUNK, _CHUNK)],
            ssems[b])

    def wait_scatter(b):
        pltpu.make_async_copy(
            rows_v.at[b], out_hbm.at[pl.ds(base, _CHUNK)], ssems[b]).wait()

    for j in range(_AHEAD):
        start_gather(j, j % _NBUF)

    def ring(g):
        for b0 in range(_NBUF):
            i = g + b0
            b = b0
            nb = (b0 + _AHEAD) % _NBUF

            @pl.when(i >= _NBUF - _AHEAD)
            def _():
                wait_scatter(nb)

            @pl.when(i + _AHEAD < _N_CHUNKS)
            def _():
                start_gather(i + _AHEAD, nb)

            wait_gather(b)
            start_scatter(i, b)

    pl.loop(0, _N_CHUNKS, step=_NBUF)(ring)

    for j in range(_NBUF - _AHEAD):
        wait_scatter((_N_CHUNKS - 1 - j) % _NBUF)


@functools.partial(
    pl.kernel,
    out_type=jax.ShapeDtypeStruct((NUM_TOKENS, D_MODEL), jnp.float32),
    mesh=plsc.VectorSubcoreMesh(core_axis_name="c", subcore_axis_name="s"),
    scratch_types=[
        pltpu.VMEM((_B_PER_W,), jnp.int32),
        pltpu.VMEM((_NBUF, _CHUNK, D_MODEL), jnp.float32),
    ] + [pltpu.SemaphoreType.DMA] * (2 * _NBUF),
)
def _sc_gather(ids_hbm, table_hbm, out_hbm, idx_v, rows_v, *sems):
    _gather_body(ids_hbm, table_hbm, out_hbm, idx_v, rows_v, *sems)


def kernel(input_ids, embed_table):
    ids_flat = jnp.reshape(input_ids, (NUM_TOKENS,)).astype(jnp.int32)
    out = _sc_gather(ids_flat, embed_table)
    return jnp.reshape(out, (*input_ids.shape, D_MODEL))
